# Initial kernel scaffold; baseline (speedup 1.0000x reference)
import jax
import jax.numpy as jnp
from jax.experimental import pallas as pl


def kernel(feature, Wc, bc, Wcls, bcls, Wreg, breg):
    raise NotImplementedError("write your pallas kernel here")


# speedup vs baseline: 44.2535x; 44.2535x over previous
import jax
import jax.numpy as jnp
import numpy as np
from jax.experimental import pallas as pl
from jax.experimental.pallas import tpu as pltpu

B, C, H, W = 2, 512, 32, 32
A = 9
N_PROP = H * W * A
NR = N_PROP // 128
PRE_NMS = 2000
POST_NMS = 2000
NSORT = 2048
NMS_THRESH = 0.7
BASE_ANCHORS = np.array([[128.,128.],[256.,256.],[512.,512.],[181.,90.],[362.,181.],[724.,362.],[90.,181.],[181.,362.],[362.,724.]], dtype=np.float32)


def _dot32(a, b):
    return jax.lax.dot(a, b, preferred_element_type=jnp.float32)


def _rank_kernel(s_ref, v_ref, out_ref):
    sm = s_ref[0]
    lane = jax.lax.broadcasted_iota(jnp.int32, (1, 128), 1)
    sub = jax.lax.broadcasted_iota(jnp.int32, (128, 1), 0)
    diag_tie = lane < sub
    st = sm.T
    cols = []
    for ic in range(NR):
        ci = st[:, ic:ic + 1]
        acc = jnp.zeros((128, 128), jnp.float32)
        for jc in range(NR):
            rj = sm[jc:jc + 1, :]
            if jc < ic:
                cmp = (rj >= ci).astype(jnp.float32)
            elif jc > ic:
                cmp = (rj > ci).astype(jnp.float32)
            else:
                cmp = jnp.where(diag_tie,
                                (rj >= ci).astype(jnp.float32),
                                (rj > ci).astype(jnp.float32))
            acc = acc + cmp
        cols.append(jnp.sum(acc, axis=1, keepdims=True))
    rank_cols = jnp.concatenate(
        cols + [jnp.zeros((128, 128 - NR), jnp.float32)], axis=1)
    ranks = rank_cols.T
    for rt in range(NSORT // 128):
        tgt = sub.astype(jnp.float32) + (rt * 128)
        accv = jnp.zeros((128, 16), jnp.float32)
        for jc in range(NR):
            rrow = ranks[jc:jc + 1, :]
            oh = (tgt == rrow).astype(jnp.bfloat16)
            coords = v_ref[0, :, jc, :]
            hp = coords.astype(jnp.bfloat16).astype(jnp.float32)
            rr = coords - hp
            mp = rr.astype(jnp.bfloat16).astype(jnp.float32)
            lp = rr - mp
            vp = jnp.concatenate(
                [hp, mp, lp, jnp.zeros((4, 128), jnp.float32)], axis=0)
            accv = accv + jax.lax.dot_general(
                oh, vp.astype(jnp.bfloat16),
                (((1,), (1,)), ((), ())), preferred_element_type=jnp.float32)
        x1 = accv[:, 0:1] + accv[:, 4:5] + accv[:, 8:9]
        y1 = accv[:, 1:2] + accv[:, 5:6] + accv[:, 9:10]
        x2 = accv[:, 2:3] + accv[:, 6:7] + accv[:, 10:11]
        y2 = accv[:, 3:4] + accv[:, 7:8] + accv[:, 11:12]
        out_ref[0, rt] = jnp.concatenate(
            [x1, y1, x2, y2, jnp.zeros((128, 4), jnp.float32)], axis=1)


def _rank_scatter(s128, vsplit):
    return pl.pallas_call(
        _rank_kernel,
        grid=(B,),
        in_specs=[pl.BlockSpec((1, 128, 128), lambda b: (b, 0, 0)),
                  pl.BlockSpec((1, 4, 128, 128), lambda b: (b, 0, 0, 0))],
        out_specs=pl.BlockSpec((1, NSORT // 128, 128, 8), lambda b: (b, 0, 0, 0)),
        out_shape=jax.ShapeDtypeStruct((B, NSORT // 128, 128, 8), jnp.float32),
    )(s128, vsplit)


def _nms_kernel(x1s_ref, y1s_ref, x2s_ref, y2s_ref, bv_ref, out_ref):
    x1v = bv_ref[0, 0]
    y1v = bv_ref[0, 1]
    x2v = bv_ref[0, 2]
    y2v = bv_ref[0, 3]
    areas = (x2v - x1v) * (y2v - y1v)
    iota = (jax.lax.broadcasted_iota(jnp.int32, (16, 128), 0) * 128
            + jax.lax.broadcasted_iota(jnp.int32, (16, 128), 1)).astype(jnp.float32)

    def body(i, keep):
        x1i = x1s_ref[0, 0, i]
        y1i = y1s_ref[0, 0, i]
        x2i = x2s_ref[0, 0, i]
        y2i = y2s_ref[0, 0, i]
        ai = (x2i - x1i) * (y2i - y1i)
        xx1 = jnp.maximum(x1i, x1v)
        yy1 = jnp.maximum(y1i, y1v)
        xx2 = jnp.minimum(x2i, x2v)
        yy2 = jnp.minimum(y2i, y2v)
        inter = jnp.maximum(xx2 - xx1, 0.0) * jnp.maximum(yy2 - yy1, 0.0)
        iou = inter / (ai + areas - inter + 1e-7)
        fi = i.astype(jnp.float32)
        sup = jnp.where((iou > NMS_THRESH) & (iota > fi), 1.0, 0.0)
        ki = jnp.sum(keep * jnp.where(iota == fi, 1.0, 0.0))
        return keep * (1.0 - sup * ki)

    keep = jax.lax.fori_loop(0, PRE_NMS, body, jnp.ones((16, 128), jnp.float32))
    keep = keep * jnp.where(iota < float(PRE_NMS), 1.0, 0.0)
    l0 = jax.lax.broadcasted_iota(jnp.int32, (128, 128), 0)
    l1 = jax.lax.broadcasted_iota(jnp.int32, (128, 128), 1)
    ut = (l0 <= l1).astype(jnp.bfloat16)
    rowcum = _dot32(keep.astype(jnp.bfloat16), ut)
    rowtot = rowcum[:, 127:128]
    r0 = jax.lax.broadcasted_iota(jnp.int32, (16, 16), 0)
    r1 = jax.lax.broadcasted_iota(jnp.int32, (16, 16), 1)
    lt16 = (r1 < r0).astype(jnp.bfloat16)
    rowoff = _dot32(lt16, rowtot.astype(jnp.bfloat16))
    pos = rowcum + rowoff - 1.0
    cx = (x1v + x2v) / 2
    cy = (y1v + y2v) / 2
    bw = x2v - x1v
    bh = y2v - y1v
    splits = []
    for v in (cx, cy, bw, bh):
        hpart = v.astype(jnp.bfloat16).astype(jnp.float32)
        r = v - hpart
        mpart = r.astype(jnp.bfloat16).astype(jnp.float32)
        lpart = r - mpart
        splits.extend([hpart, mpart, lpart])
    sub = jax.lax.broadcasted_iota(jnp.int32, (128, 1), 0)
    for rt in range(NSORT // 128):
        tgt = sub.astype(jnp.float32) + (rt * 128)
        accv = jnp.zeros((128, 16), jnp.float32)
        for tc in range(NSORT // 128):
            prow = pos[tc:tc + 1, :]
            krow = keep[tc:tc + 1, :]
            oh = ((tgt == prow).astype(jnp.float32)
                  * (krow > 0.0).astype(jnp.float32)).astype(jnp.bfloat16)
            vp = jnp.concatenate(
                [sp[tc:tc + 1, :] for sp in splits]
                + [jnp.zeros((4, 128), jnp.float32)], axis=0).astype(jnp.bfloat16)
            accv = accv + jax.lax.dot_general(
                oh, vp, (((1,), (1,)), ((), ())),
                preferred_element_type=jnp.float32)
        xo = accv[:, 0:1] + accv[:, 1:2] + accv[:, 2:3]
        yo = accv[:, 3:4] + accv[:, 4:5] + accv[:, 5:6]
        wo = accv[:, 6:7] + accv[:, 7:8] + accv[:, 8:9]
        ho = accv[:, 9:10] + accv[:, 10:11] + accv[:, 11:12]
        out_ref[0, rt] = jnp.concatenate(
            [xo, yo, wo, ho, jnp.zeros((128, 4), jnp.float32)], axis=1)


def _nms(coords_s, bs_vec):
    smem_spec = pl.BlockSpec((1, 1, NSORT), lambda b: (b, 0, 0),
                             memory_space=pltpu.SMEM)
    return pl.pallas_call(
        _nms_kernel,
        grid=(B,),
        in_specs=[smem_spec, smem_spec, smem_spec, smem_spec,
                  pl.BlockSpec((1, 4, 16, 128), lambda b: (b, 0, 0, 0))],
        out_specs=pl.BlockSpec((1, NSORT // 128, 128, 8), lambda b: (b, 0, 0, 0)),
        out_shape=jax.ShapeDtypeStruct((B, NSORT // 128, 128, 8), jnp.float32),
    )(*coords_s, bs_vec)


def _conv2d(x, w, b, pad):
    out = jax.lax.conv_general_dilated(
        x, w, window_strides=(1, 1), padding=[(pad, pad), (pad, pad)],
        dimension_numbers=('NCHW', 'OIHW', 'NCHW'))
    return out + b[None, :, None, None]


def _forward_props(feature, Wc, bc, Wcls, bcls, Wreg, breg):
    x = jax.nn.relu(_conv2d(feature, Wc, bc, 1))
    cls = _conv2d(x, Wcls, bcls, 0).transpose(0, 2, 3, 1).reshape(B, H, W, A, 2)
    reg = _conv2d(x, Wreg, breg, 0).transpose(0, 2, 3, 1).reshape(B, H, W, A, 4)
    wh = jnp.asarray(BASE_ANCHORS) / 16.0
    anchors_wh = jnp.broadcast_to(wh[None, None, None, :, :], (1, H, W, A, 2))
    gx, gy = jnp.meshgrid(jnp.arange(W, dtype=jnp.float32),
                          jnp.arange(H, dtype=jnp.float32), indexing='xy')
    xy = jnp.stack([gx, gy], axis=-1)
    anchors_xy = jnp.broadcast_to(xy[None, :, :, None, :], (1, H, W, A, 2))
    anchors = jnp.concatenate([anchors_xy, anchors_wh], axis=-1)
    d = reg
    a = anchors
    px = d[..., 0] * a[..., 2] + a[..., 0]
    py = d[..., 1] * a[..., 3] + a[..., 1]
    pw = jnp.exp(d[..., 2]) * a[..., 2]
    ph = jnp.exp(d[..., 2]) * a[..., 3]
    xywh = jnp.stack([px, py, pw, ph], axis=-1)
    score = jax.nn.softmax(cls, axis=4)[..., 1]
    props = jnp.concatenate([score[..., None], xywh], axis=4).reshape(B, -1, 5)
    x1 = props[..., 1] - props[..., 3] / 2
    y1 = props[..., 2] - props[..., 4] / 2
    x2 = props[..., 1] + props[..., 3] / 2
    y2 = props[..., 2] + props[..., 4] / 2
    bx1 = jnp.clip(x1, 0.0, W - 1.0)
    by1 = jnp.clip(y1, 0.0, H - 1.0)
    bx2 = jnp.clip(x2, 0.0, W - 1.0)
    by2 = jnp.clip(y2, 0.0, H - 1.0)
    return props[..., 0], jnp.stack([bx1, by1, bx2, by2], axis=-1)


def kernel(feature, Wc, bc, Wcls, bcls, Wreg, breg):
    scores, boxes = _forward_props(
        jax.lax.stop_gradient(feature), Wc, bc, Wcls, bcls, Wreg, breg)
    s72 = scores.reshape(B, NR, 128)
    s128 = jnp.pad(s72, ((0, 0), (0, 128 - NR), (0, 0)),
                   constant_values=-jnp.inf)
    vraw = jnp.stack([boxes[..., ci].reshape(B, NR, 128) for ci in range(4)],
                     axis=1)
    vraw = jnp.pad(vraw, ((0, 0), (0, 0), (0, 128 - NR), (0, 0)))

    bs = _rank_scatter(s128, vraw)
    bs_flat = bs.reshape(B, NSORT, 8)
    bs_vec = jnp.transpose(bs_flat[:, :, :4], (0, 2, 1)).reshape(
        B, 4, NSORT // 128, 128)

    coords_s = [bs_flat[:, :, ci].reshape(B, 1, NSORT) for ci in range(4)]
    out = _nms(coords_s, bs_vec)
    return out.reshape(B, NSORT, 8)[:, :POST_NMS, :4]
